# baseline (speedup 1.0000x reference)
import functools
import math

import jax
import jax.numpy as jnp
from jax.experimental import pallas as pl
from jax.experimental.pallas import tpu as pltpu

B, S = 4, 2048
DM, H, DK, DV, DI = 512, 4, 128, 128, 1024
NT, NV = 64, 5000
NVP = 5120
TI = 256
NBLK = S // TI

_HALF_PI = math.pi / 2.0


def _sin_poly(x):
    k = jnp.round(x * (1.0 / _HALF_PI))
    y = x - k * _HALF_PI
    y2 = y * y
    s = y * (1.0 + y2 * (-1.0 / 6.0 + y2 * (1.0 / 120.0 + y2 * (-1.0 / 5040.0))))
    c = 1.0 + y2 * (-0.5 + y2 * (1.0 / 24.0 + y2 * (-1.0 / 720.0)))
    ki = k - 4.0 * jnp.floor(k * 0.25)
    use_cos = jnp.logical_or(ki == 1.0, ki == 3.0)
    neg = jnp.logical_or(ki == 2.0, ki == 3.0)
    r = jnp.where(use_cos, c, s)
    return jnp.where(neg, -r, r)


CH0 = 250


def _k0_body(a_ref, w_ref, p_ref):
    p_ref[...] = a_ref[...] * w_ref[...] * 10.0


def _run_k0(A3, W3):
    return pl.pallas_call(
        _k0_body,
        grid=(NV // CH0,),
        in_specs=[
            pl.BlockSpec((CH0, 1, NV), lambda i: (i, 0, 0)),
            pl.BlockSpec((CH0, 1, NV), lambda i: (i, 0, 0)),
        ],
        out_specs=pl.BlockSpec((CH0, 1, NV), lambda i: (i, 0, 0)),
        out_shape=jax.ShapeDtypeStruct((NV, 1, NV), jnp.float32),
        compiler_params=pltpu.CompilerParams(
            dimension_semantics=("arbitrary",),
            vmem_limit_bytes=56 * 1024 * 1024,
        ),
        name="k0_p_prepass",
    )(A3, W3)


RPB = 64
NSLOT = 4
GK1 = (B * S) // RPB


def _k1_body(ridx_s, et_s, vt_s, p_any, ee_ref, ve_ref,
             r_ref, e_ref, pbuf, sems):
    ii = pl.program_id(0)
    base = ii * RPB

    def issue(slot, rowbase):
        for mi in range(RPB):
            t = jnp.minimum(rowbase + mi, B * S - 1)
            pltpu.make_async_copy(p_any.at[ridx_s[t]],
                                  pbuf.at[slot * RPB + mi],
                                  sems.at[slot]).start(priority=mi % 2)

    @pl.when(ii == 0)
    def _():
        for d in range(NSLOT - 1):
            issue(d, base + d * RPB)

    slot = jax.lax.rem(ii, NSLOT)

    @pl.when(ii < GK1 - (NSLOT - 1))
    def _():
        issue(jax.lax.rem(ii + NSLOT - 1, NSLOT), base + (NSLOT - 1) * RPB)

    for mi in range(RPB):
        t = base + mi
        e_ref[mi, 0] = ee_ref[et_s[t], 0] + ve_ref[vt_s[t], 0]

    sl = pl.ds(slot * RPB, RPB)
    pltpu.make_async_copy(pbuf.at[sl], pbuf.at[sl], sems.at[slot]).wait()
    r_ref[...] = pbuf[sl].astype(jnp.bfloat16)


def _run_k1(ridx, etf, vtf, P3, EE, VE):
    grid_spec = pltpu.PrefetchScalarGridSpec(
        num_scalar_prefetch=3,
        grid=(GK1,),
        in_specs=[
            pl.BlockSpec(memory_space=pl.ANY),
            pl.BlockSpec((NT + 1, 1, DM), lambda i, r, e, v: (0, 0, 0)),
            pl.BlockSpec((NV + 1, 1, DM), lambda i, r, e, v: (0, 0, 0)),
        ],
        out_specs=[
            pl.BlockSpec((RPB, 1, NV), lambda i, r, e, v: (i, 0, 0)),
            pl.BlockSpec((RPB, 1, DM), lambda i, r, e, v: (i, 0, 0)),
        ],
        scratch_shapes=[
            pltpu.VMEM((NSLOT * RPB, 1, NV), jnp.float32),
            pltpu.SemaphoreType.DMA((NSLOT,)),
        ],
    )
    return pl.pallas_call(
        _k1_body,
        grid_spec=grid_spec,
        out_shape=[
            jax.ShapeDtypeStruct((B * S, 1, NV), jnp.bfloat16),
            jax.ShapeDtypeStruct((B * S, 1, DM), jnp.float32),
        ],
        compiler_params=pltpu.CompilerParams(
            dimension_semantics=("arbitrary",),
            vmem_limit_bytes=56 * 1024 * 1024,
        ),
        name="k1_row_gather",
    )(ridx, etf, vtf, P3, EE, VE)


NVCH = 720


def _build_oh_rows(oh_slot, vrow, lo, n):
    idxr = jnp.maximum(vrow - 1, 0)
    cio = jax.lax.broadcasted_iota(jnp.int32, (n, S), 0) + lo
    oh_slot[pl.ds(lo, n)] = jnp.where(cio == idxr, 1.0, 0.0).astype(
        jnp.bfloat16)


def _k2_body(vert_ref, vertn_ref, vmr_ref, r_ref, sim_ref, oh_ref):
    b = pl.program_id(0)
    i = pl.program_id(1)
    cur = jax.lax.rem(b, 2)

    @pl.when(jnp.logical_and(b == 0, i == 0))
    def _():
        _build_oh_rows(oh_ref.at[0], vert_ref[0], 0, NV)

    @pl.when(jnp.logical_and(b < B - 1, i > 0))
    def _():
        lo = jnp.minimum((i - 1) * NVCH, NV - NVCH)
        _build_oh_rows(oh_ref.at[1 - cur], vertn_ref[0],
                       pl.multiple_of(lo, 8), NVCH)

    simb = jnp.dot(r_ref[0], oh_ref[cur],
                   preferred_element_type=jnp.float32)
    colmask = jnp.where(vert_ref[0] != 0, 1.0, 0.0)
    sim_ref[0] = simb * (colmask * vmr_ref[0])


def _run_k2(vert3, vmcol, Rb):
    return pl.pallas_call(
        _k2_body,
        grid=(B, NBLK),
        in_specs=[
            pl.BlockSpec((1, 1, S), lambda b, i: (b, 0, 0)),
            pl.BlockSpec((1, 1, S), lambda b, i: (jnp.minimum(b + 1, B - 1), 0, 0)),
            pl.BlockSpec((1, TI, 1), lambda b, i: (b, i, 0)),
            pl.BlockSpec((1, TI, NV), lambda b, i: (b, i, 0)),
        ],
        out_specs=pl.BlockSpec((1, TI, S), lambda b, i: (b, i, 0)),
        out_shape=jax.ShapeDtypeStruct((B, S, S), jnp.float32),
        scratch_shapes=[pltpu.VMEM((2, NV, S), jnp.bfloat16)],
        compiler_params=pltpu.CompilerParams(
            dimension_semantics=("arbitrary", "arbitrary"),
            vmem_limit_bytes=56 * 1024 * 1024,
        ),
        name="k2_sim",
    )(vert3, vert3, vmcol, Rb)


def _k3_body(enc_full_ref, sim_ref, et_ref, t_ref, np_ref, npb_ref,
             wq_ref, wk_ref, wv_ref, wo_ref, bo_ref,
             g1_ref, b1l_ref, w1_ref, b1_ref, w2_ref, b2_ref,
             g2_ref, b2l_ref, ipv_ref, odd_ref,
             out_ref, ef_s, k_s, v_s):
    i = pl.program_id(1)

    @pl.when(i == 0)
    def _():
        tcol = t_ref[0]
        npcol = np_ref[0]
        arg = tcol * ipv_ref[...] + odd_ref[...]
        tem = _sin_poly(arg) * npcol
        ef = enc_full_ref[0] + tem
        ef_s[...] = ef
        eb = ef.astype(jnp.bfloat16)
        k_s[...] = jnp.dot(eb, wk_ref[...],
                           preferred_element_type=jnp.float32).astype(jnp.bfloat16)
        v_s[...] = jnp.dot(eb, wv_ref[...],
                           preferred_element_type=jnp.float32).astype(jnp.bfloat16)

    x = ef_s[pl.ds(i * TI, TI), :]
    q = jnp.dot(x.astype(jnp.bfloat16), wq_ref[...],
                preferred_element_type=jnp.float32)
    qb = (q * (1.0 / math.sqrt(DK))).astype(jnp.bfloat16)

    colg = jax.lax.broadcasted_iota(jnp.int32, (TI, S), 1)
    rowg = jax.lax.broadcasted_iota(jnp.int32, (TI, S), 0) + i * TI
    mask = jnp.logical_or(et_ref[0] == 0, colg > rowg)
    simbias = jnp.where(mask, -1e9, sim_ref[0])

    outs = []
    for h in range(H):
        hs = slice(h * DK, (h + 1) * DK)
        sh = jax.lax.dot_general(
            qb[:, hs], k_s[:, hs],
            (((1,), (1,)), ((), ())),
            preferred_element_type=jnp.float32,
        ) + simbias
        m = jnp.max(sh, axis=-1, keepdims=True)
        p = jnp.exp(sh - m)
        l = jnp.sum(p, axis=-1, keepdims=True)
        oh = jnp.dot(p.astype(jnp.bfloat16), v_s[:, hs],
                     preferred_element_type=jnp.float32)
        outs.append(oh * (1.0 / l))
    o = jnp.concatenate(outs, axis=-1)
    o = jnp.dot(o.astype(jnp.bfloat16), wo_ref[...],
                preferred_element_type=jnp.float32) + bo_ref[...]

    npc = npb_ref[0]
    x1 = o + x
    mu = jnp.mean(x1, axis=-1, keepdims=True)
    d = x1 - mu
    var = jnp.mean(d * d, axis=-1, keepdims=True)
    y = d * jax.lax.rsqrt(var + 1e-5) * g1_ref[...] + b1l_ref[...]
    y = y * npc

    h1 = jnp.dot(y.astype(jnp.bfloat16), w1_ref[...],
                 preferred_element_type=jnp.float32) + b1_ref[...]
    h1 = jnp.maximum(h1, 0.0)
    f = jnp.dot(h1.astype(jnp.bfloat16), w2_ref[...],
                preferred_element_type=jnp.float32) + b2_ref[...]
    x2 = f + y
    mu2 = jnp.mean(x2, axis=-1, keepdims=True)
    d2 = x2 - mu2
    var2 = jnp.mean(d2 * d2, axis=-1, keepdims=True)
    z = d2 * jax.lax.rsqrt(var2 + 1e-5) * g2_ref[...] + b2l_ref[...]
    out_ref[0] = z * npc


def _run_k3(enc3, sim, et3, t3, np3, weights):
    (wq, wk, wv, wo, bo, g1, b1l, w1, b1, w2, b2, g2, b2l, ipv, odd) = weights
    full = lambda b, i: (b, 0, 0)
    blk = lambda b, i: (b, i, 0)
    zz = lambda b, i: (0, 0)
    return pl.pallas_call(
        _k3_body,
        grid=(B, NBLK),
        in_specs=[
            pl.BlockSpec((1, S, DM), full),
            pl.BlockSpec((1, TI, S), blk),
            pl.BlockSpec((1, 1, S), full),
            pl.BlockSpec((1, S, 1), full),
            pl.BlockSpec((1, S, 1), full),
            pl.BlockSpec((1, TI, 1), blk),
            pl.BlockSpec((DM, DM), zz),
            pl.BlockSpec((DM, DM), zz),
            pl.BlockSpec((DM, DM), zz),
            pl.BlockSpec((DM, DM), zz),
            pl.BlockSpec((1, DM), zz),
            pl.BlockSpec((1, DM), zz),
            pl.BlockSpec((1, DM), zz),
            pl.BlockSpec((DM, DI), zz),
            pl.BlockSpec((1, DI), zz),
            pl.BlockSpec((DI, DM), zz),
            pl.BlockSpec((1, DM), zz),
            pl.BlockSpec((1, DM), zz),
            pl.BlockSpec((1, DM), zz),
            pl.BlockSpec((1, DM), zz),
            pl.BlockSpec((1, DM), zz),
        ],
        out_specs=pl.BlockSpec((1, TI, DM), blk),
        out_shape=jax.ShapeDtypeStruct((B, S, DM), jnp.float32),
        scratch_shapes=[
            pltpu.VMEM((S, DM), jnp.float32),
            pltpu.VMEM((S, DM), jnp.bfloat16),
            pltpu.VMEM((S, DM), jnp.bfloat16),
        ],
        compiler_params=pltpu.CompilerParams(
            dimension_semantics=("parallel", "arbitrary"),
            vmem_limit_bytes=56 * 1024 * 1024,
        ),
        name="k3_encoder",
    )(enc3, sim, et3, t3, np3, np3,
      wq, wk, wv, wo, bo, g1, b1l, w1, b1, w2, b2, g2, b2l, ipv, odd)


def kernel(event_type, vertex, event_time, non_pad_mask, A, W, event_emb,
           vertex_emb, Wq, Wk, Wv, Wo, bo, ln1_g, ln1_b, w1, b1, w2, b2,
           ln2_g, ln2_b):
    et = event_type.astype(jnp.int32)
    vt = vertex.astype(jnp.int32)
    ridx = jnp.clip(vt - 1, 0).reshape(-1)
    etf = et.reshape(-1)
    vtf = vt.reshape(-1)
    A3 = A.reshape(NV, 1, NV)
    W3 = W.reshape(NV, 1, NV)
    EE = event_emb.reshape(NT + 1, 1, DM)
    VE = vertex_emb.reshape(NV + 1, 1, DM)

    P3 = _run_k0(A3, W3)
    R, enc0 = _run_k1(ridx, etf, vtf, P3, EE, VE)

    vert3 = vt.reshape(B, 1, S)
    vmcol = (vt != 0).astype(jnp.float32).reshape(B, S, 1)
    sim = _run_k2(vert3, vmcol, R.reshape(B, S, NV))

    iarr = jnp.arange(DM)
    pv = jnp.power(jnp.float32(10000.0),
                   2.0 * (iarr // 2).astype(jnp.float32) / DM)
    ipv = (1.0 / pv).reshape(1, DM)
    odd = jnp.where(iarr % 2 == 0, 0.0, _HALF_PI).astype(
        jnp.float32).reshape(1, DM)

    bf = jnp.bfloat16
    weights = (Wq.astype(bf), Wk.astype(bf), Wv.astype(bf), Wo.astype(bf),
               bo.reshape(1, DM), ln1_g.reshape(1, DM), ln1_b.reshape(1, DM),
               w1.astype(bf), b1.reshape(1, DI), w2.astype(bf),
               b2.reshape(1, DM), ln2_g.reshape(1, DM), ln2_b.reshape(1, DM),
               ipv, odd)

    et3 = et.reshape(B, 1, S)
    t3 = event_time.astype(jnp.float32).reshape(B, S, 1)
    np3 = (et != 0).astype(jnp.float32).reshape(B, S, 1)
    enc = _run_k3(enc0.reshape(B, S, DM), sim, et3, t3, np3, weights)

    return enc, sim.reshape(B, 1, S, S)


# speedup vs baseline: 1.0272x; 1.0272x over previous
import functools
import math

import jax
import jax.numpy as jnp
from jax.experimental import pallas as pl
from jax.experimental.pallas import tpu as pltpu

B, S = 4, 2048
DM, H, DK, DV, DI = 512, 4, 128, 128, 1024
NT, NV = 64, 5000
NVP = 5120
TI = 256
NBLK = S // TI
TI3 = 512
NBLK3 = S // TI3

_HALF_PI = math.pi / 2.0


def _sin_poly(x):
    k = jnp.round(x * (1.0 / _HALF_PI))
    y = x - k * _HALF_PI
    y2 = y * y
    s = y * (1.0 + y2 * (-1.0 / 6.0 + y2 * (1.0 / 120.0 + y2 * (-1.0 / 5040.0))))
    c = 1.0 + y2 * (-0.5 + y2 * (1.0 / 24.0 + y2 * (-1.0 / 720.0)))
    ki = k - 4.0 * jnp.floor(k * 0.25)
    use_cos = jnp.logical_or(ki == 1.0, ki == 3.0)
    neg = jnp.logical_or(ki == 2.0, ki == 3.0)
    r = jnp.where(use_cos, c, s)
    return jnp.where(neg, -r, r)


CH0 = 250


def _k0_body(a_ref, w_ref, p_ref):
    p_ref[...] = a_ref[...] * w_ref[...] * 10.0


def _run_k0(A3, W3):
    return pl.pallas_call(
        _k0_body,
        grid=(NV // CH0,),
        in_specs=[
            pl.BlockSpec((CH0, 1, NV), lambda i: (i, 0, 0)),
            pl.BlockSpec((CH0, 1, NV), lambda i: (i, 0, 0)),
        ],
        out_specs=pl.BlockSpec((CH0, 1, NV), lambda i: (i, 0, 0)),
        out_shape=jax.ShapeDtypeStruct((NV, 1, NV), jnp.float32),
        compiler_params=pltpu.CompilerParams(
            dimension_semantics=("arbitrary",),
            vmem_limit_bytes=56 * 1024 * 1024,
        ),
        name="k0_p_prepass",
    )(A3, W3)


RPB = 64
NSLOT = 4
GK1 = (B * S) // RPB


def _k1_body(ridx_s, et_s, vt_s, p_any, ee_ref, ve_ref,
             r_ref, e_ref, pbuf, sems):
    ii = pl.program_id(0)
    base = ii * RPB

    def issue(slot, rowbase):
        for mi in range(RPB):
            t = jnp.minimum(rowbase + mi, B * S - 1)
            pltpu.make_async_copy(p_any.at[ridx_s[t]],
                                  pbuf.at[slot * RPB + mi],
                                  sems.at[slot]).start()

    @pl.when(ii == 0)
    def _():
        for d in range(NSLOT - 1):
            issue(d, base + d * RPB)

    slot = jax.lax.rem(ii, NSLOT)

    @pl.when(ii < GK1 - (NSLOT - 1))
    def _():
        issue(jax.lax.rem(ii + NSLOT - 1, NSLOT), base + (NSLOT - 1) * RPB)

    for mi in range(RPB):
        t = base + mi
        e_ref[mi, 0] = ee_ref[et_s[t], 0] + ve_ref[vt_s[t], 0]

    sl = pl.ds(slot * RPB, RPB)
    pltpu.make_async_copy(pbuf.at[sl], pbuf.at[sl], sems.at[slot]).wait()
    r_ref[...] = pbuf[sl].astype(jnp.bfloat16)


def _run_k1(ridx, etf, vtf, P3, EE, VE):
    grid_spec = pltpu.PrefetchScalarGridSpec(
        num_scalar_prefetch=3,
        grid=(GK1,),
        in_specs=[
            pl.BlockSpec(memory_space=pl.ANY),
            pl.BlockSpec((NT + 1, 1, DM), lambda i, r, e, v: (0, 0, 0)),
            pl.BlockSpec((NV + 1, 1, DM), lambda i, r, e, v: (0, 0, 0)),
        ],
        out_specs=[
            pl.BlockSpec((RPB, 1, NV), lambda i, r, e, v: (i, 0, 0)),
            pl.BlockSpec((RPB, 1, DM), lambda i, r, e, v: (i, 0, 0)),
        ],
        scratch_shapes=[
            pltpu.VMEM((NSLOT * RPB, 1, NV), jnp.float32),
            pltpu.SemaphoreType.DMA((NSLOT,)),
        ],
    )
    return pl.pallas_call(
        _k1_body,
        grid_spec=grid_spec,
        out_shape=[
            jax.ShapeDtypeStruct((B * S, 1, NV), jnp.bfloat16),
            jax.ShapeDtypeStruct((B * S, 1, DM), jnp.float32),
        ],
        compiler_params=pltpu.CompilerParams(
            dimension_semantics=("arbitrary",),
            vmem_limit_bytes=56 * 1024 * 1024,
        ),
        name="k1_row_gather",
    )(ridx, etf, vtf, P3, EE, VE)


NVCH = 720


def _build_oh_rows(oh_slot, vrow, lo, n):
    idxr = jnp.maximum(vrow - 1, 0)
    cio = jax.lax.broadcasted_iota(jnp.int32, (n, S), 0) + lo
    oh_slot[pl.ds(lo, n)] = jnp.where(cio == idxr, 1.0, 0.0).astype(
        jnp.bfloat16)


def _k2_body(vert_ref, vertn_ref, vmr_ref, r_ref, sim_ref, oh_ref):
    b = pl.program_id(0)
    i = pl.program_id(1)
    cur = jax.lax.rem(b, 2)

    @pl.when(jnp.logical_and(b == 0, i == 0))
    def _():
        _build_oh_rows(oh_ref.at[0], vert_ref[0], 0, NV)

    @pl.when(jnp.logical_and(b < B - 1, i > 0))
    def _():
        lo = jnp.minimum((i - 1) * NVCH, NV - NVCH)
        _build_oh_rows(oh_ref.at[1 - cur], vertn_ref[0],
                       pl.multiple_of(lo, 8), NVCH)

    simb = jnp.dot(r_ref[0], oh_ref[cur],
                   preferred_element_type=jnp.float32)
    colmask = jnp.where(vert_ref[0] != 0, 1.0, 0.0)
    sim_ref[0] = simb * (colmask * vmr_ref[0])


def _run_k2(vert3, vmcol, Rb):
    return pl.pallas_call(
        _k2_body,
        grid=(B, NBLK),
        in_specs=[
            pl.BlockSpec((1, 1, S), lambda b, i: (b, 0, 0)),
            pl.BlockSpec((1, 1, S), lambda b, i: (jnp.minimum(b + 1, B - 1), 0, 0)),
            pl.BlockSpec((1, TI, 1), lambda b, i: (b, i, 0)),
            pl.BlockSpec((1, TI, NV), lambda b, i: (b, i, 0)),
        ],
        out_specs=pl.BlockSpec((1, TI, S), lambda b, i: (b, i, 0)),
        out_shape=jax.ShapeDtypeStruct((B, S, S), jnp.float32),
        scratch_shapes=[pltpu.VMEM((2, NV, S), jnp.bfloat16)],
        compiler_params=pltpu.CompilerParams(
            dimension_semantics=("arbitrary", "arbitrary"),
            vmem_limit_bytes=56 * 1024 * 1024,
        ),
        name="k2_sim",
    )(vert3, vert3, vmcol, Rb)


def _k3_body(enc_full_ref, sim_ref, et_ref, t_ref, np_ref, npb_ref,
             wq_ref, wk_ref, wv_ref, wo_ref, bo_ref,
             g1_ref, b1l_ref, w1_ref, b1_ref, w2_ref, b2_ref,
             g2_ref, b2l_ref, ipv_ref, odd_ref,
             out_ref, ef_s, k_s, v_s):
    i = pl.program_id(1)

    @pl.when(i == 0)
    def _():
        tcol = t_ref[0]
        npcol = np_ref[0]
        arg = tcol * ipv_ref[...] + odd_ref[...]
        tem = _sin_poly(arg) * npcol
        ef = enc_full_ref[0] + tem
        ef_s[...] = ef
        eb = ef.astype(jnp.bfloat16)
        k_s[...] = jnp.dot(eb, wk_ref[...],
                           preferred_element_type=jnp.float32).astype(jnp.bfloat16)
        v_s[...] = jnp.dot(eb, wv_ref[...],
                           preferred_element_type=jnp.float32).astype(jnp.bfloat16)

    x = ef_s[pl.ds(i * TI3, TI3), :]
    q = jnp.dot(x.astype(jnp.bfloat16), wq_ref[...],
                preferred_element_type=jnp.float32)
    qb = (q * (1.0 / math.sqrt(DK))).astype(jnp.bfloat16)

    colg = jax.lax.broadcasted_iota(jnp.int32, (TI3, S), 1)
    rowg = jax.lax.broadcasted_iota(jnp.int32, (TI3, S), 0) + i * TI3
    mask = jnp.logical_or(et_ref[0] == 0, colg > rowg)
    simbias = jnp.where(mask, -1e9, sim_ref[0])

    outs = []
    for h in range(H):
        hs = slice(h * DK, (h + 1) * DK)
        sh = jax.lax.dot_general(
            qb[:, hs], k_s[:, hs],
            (((1,), (1,)), ((), ())),
            preferred_element_type=jnp.float32,
        ) + simbias
        m = jnp.max(sh, axis=-1, keepdims=True)
        p = jnp.exp(sh - m)
        l = jnp.sum(p, axis=-1, keepdims=True)
        oh = jnp.dot(p.astype(jnp.bfloat16), v_s[:, hs],
                     preferred_element_type=jnp.float32)
        outs.append(oh * (1.0 / l))
    o = jnp.concatenate(outs, axis=-1)
    o = jnp.dot(o.astype(jnp.bfloat16), wo_ref[...],
                preferred_element_type=jnp.float32) + bo_ref[...]

    npc = npb_ref[0]
    x1 = o + x
    mu = jnp.mean(x1, axis=-1, keepdims=True)
    d = x1 - mu
    var = jnp.mean(d * d, axis=-1, keepdims=True)
    y = d * jax.lax.rsqrt(var + 1e-5) * g1_ref[...] + b1l_ref[...]
    y = y * npc

    h1 = jnp.dot(y.astype(jnp.bfloat16), w1_ref[...],
                 preferred_element_type=jnp.float32) + b1_ref[...]
    h1 = jnp.maximum(h1, 0.0)
    f = jnp.dot(h1.astype(jnp.bfloat16), w2_ref[...],
                preferred_element_type=jnp.float32) + b2_ref[...]
    x2 = f + y
    mu2 = jnp.mean(x2, axis=-1, keepdims=True)
    d2 = x2 - mu2
    var2 = jnp.mean(d2 * d2, axis=-1, keepdims=True)
    z = d2 * jax.lax.rsqrt(var2 + 1e-5) * g2_ref[...] + b2l_ref[...]
    out_ref[0] = z * npc


def _run_k3(enc3, sim, et3, t3, np3, weights):
    (wq, wk, wv, wo, bo, g1, b1l, w1, b1, w2, b2, g2, b2l, ipv, odd) = weights
    full = lambda b, i: (b, 0, 0)
    blk = lambda b, i: (b, i, 0)
    zz = lambda b, i: (0, 0)
    return pl.pallas_call(
        _k3_body,
        grid=(B, NBLK3),
        in_specs=[
            pl.BlockSpec((1, S, DM), full),
            pl.BlockSpec((1, TI3, S), blk),
            pl.BlockSpec((1, 1, S), full),
            pl.BlockSpec((1, S, 1), full),
            pl.BlockSpec((1, S, 1), full),
            pl.BlockSpec((1, TI3, 1), blk),
            pl.BlockSpec((DM, DM), zz),
            pl.BlockSpec((DM, DM), zz),
            pl.BlockSpec((DM, DM), zz),
            pl.BlockSpec((DM, DM), zz),
            pl.BlockSpec((1, DM), zz),
            pl.BlockSpec((1, DM), zz),
            pl.BlockSpec((1, DM), zz),
            pl.BlockSpec((DM, DI), zz),
            pl.BlockSpec((1, DI), zz),
            pl.BlockSpec((DI, DM), zz),
            pl.BlockSpec((1, DM), zz),
            pl.BlockSpec((1, DM), zz),
            pl.BlockSpec((1, DM), zz),
            pl.BlockSpec((1, DM), zz),
            pl.BlockSpec((1, DM), zz),
        ],
        out_specs=pl.BlockSpec((1, TI3, DM), blk),
        out_shape=jax.ShapeDtypeStruct((B, S, DM), jnp.float32),
        scratch_shapes=[
            pltpu.VMEM((S, DM), jnp.float32),
            pltpu.VMEM((S, DM), jnp.bfloat16),
            pltpu.VMEM((S, DM), jnp.bfloat16),
        ],
        compiler_params=pltpu.CompilerParams(
            dimension_semantics=("parallel", "arbitrary"),
            vmem_limit_bytes=56 * 1024 * 1024,
        ),
        name="k3_encoder",
    )(enc3, sim, et3, t3, np3, np3,
      wq, wk, wv, wo, bo, g1, b1l, w1, b1, w2, b2, g2, b2l, ipv, odd)


def kernel(event_type, vertex, event_time, non_pad_mask, A, W, event_emb,
           vertex_emb, Wq, Wk, Wv, Wo, bo, ln1_g, ln1_b, w1, b1, w2, b2,
           ln2_g, ln2_b):
    et = event_type.astype(jnp.int32)
    vt = vertex.astype(jnp.int32)
    ridx = jnp.clip(vt - 1, 0).reshape(-1)
    etf = et.reshape(-1)
    vtf = vt.reshape(-1)
    A3 = A.reshape(NV, 1, NV)
    W3 = W.reshape(NV, 1, NV)
    EE = event_emb.reshape(NT + 1, 1, DM)
    VE = vertex_emb.reshape(NV + 1, 1, DM)

    P3 = _run_k0(A3, W3)
    R, enc0 = _run_k1(ridx, etf, vtf, P3, EE, VE)

    vert3 = vt.reshape(B, 1, S)
    vmcol = (vt != 0).astype(jnp.float32).reshape(B, S, 1)
    sim = _run_k2(vert3, vmcol, R.reshape(B, S, NV))

    iarr = jnp.arange(DM)
    pv = jnp.power(jnp.float32(10000.0),
                   2.0 * (iarr // 2).astype(jnp.float32) / DM)
    ipv = (1.0 / pv).reshape(1, DM)
    odd = jnp.where(iarr % 2 == 0, 0.0, _HALF_PI).astype(
        jnp.float32).reshape(1, DM)

    bf = jnp.bfloat16
    weights = (Wq.astype(bf), Wk.astype(bf), Wv.astype(bf), Wo.astype(bf),
               bo.reshape(1, DM), ln1_g.reshape(1, DM), ln1_b.reshape(1, DM),
               w1.astype(bf), b1.reshape(1, DI), w2.astype(bf),
               b2.reshape(1, DM), ln2_g.reshape(1, DM), ln2_b.reshape(1, DM),
               ipv, odd)

    et3 = et.reshape(B, 1, S)
    t3 = event_time.astype(jnp.float32).reshape(B, S, 1)
    np3 = (et != 0).astype(jnp.float32).reshape(B, S, 1)
    enc = _run_k3(enc0.reshape(B, S, DM), sim, et3, t3, np3, weights)

    return enc, sim.reshape(B, 1, S, S)


# speedup vs baseline: 1.0305x; 1.0032x over previous
import functools
import math

import jax
import jax.numpy as jnp
from jax.experimental import pallas as pl
from jax.experimental.pallas import tpu as pltpu

B, S = 4, 2048
DM, H, DK, DV, DI = 512, 4, 128, 128, 1024
NT, NV = 64, 5000
NVP = 5120
TI = 256
NBLK = S // TI
TI3 = 512
NBLK3 = S // TI3

_HALF_PI = math.pi / 2.0


def _sin_poly(x):
    k = jnp.round(x * (1.0 / _HALF_PI))
    y = x - k * _HALF_PI
    y2 = y * y
    s = y * (1.0 + y2 * (-1.0 / 6.0 + y2 * (1.0 / 120.0 + y2 * (-1.0 / 5040.0))))
    c = 1.0 + y2 * (-0.5 + y2 * (1.0 / 24.0 + y2 * (-1.0 / 720.0)))
    ki = k - 4.0 * jnp.floor(k * 0.25)
    use_cos = jnp.logical_or(ki == 1.0, ki == 3.0)
    neg = jnp.logical_or(ki == 2.0, ki == 3.0)
    r = jnp.where(use_cos, c, s)
    return jnp.where(neg, -r, r)


CH0 = 250


def _k0_body(a_ref, w_ref, p_ref):
    p_ref[...] = a_ref[...] * w_ref[...] * 10.0


def _run_k0(A3, W3):
    return pl.pallas_call(
        _k0_body,
        grid=(NV // CH0,),
        in_specs=[
            pl.BlockSpec((CH0, 1, NV), lambda i: (i, 0, 0)),
            pl.BlockSpec((CH0, 1, NV), lambda i: (i, 0, 0)),
        ],
        out_specs=pl.BlockSpec((CH0, 1, NV), lambda i: (i, 0, 0)),
        out_shape=jax.ShapeDtypeStruct((NV, 1, NV), jnp.float32),
        compiler_params=pltpu.CompilerParams(
            dimension_semantics=("arbitrary",),
            vmem_limit_bytes=56 * 1024 * 1024,
        ),
        name="k0_p_prepass",
    )(A3, W3)


RPB = 64
NSLOT = 4
GK1 = (B * S) // RPB


def _k1_body(ridx_s, et_s, vt_s, p_any, ee_ref, ve_ref,
             r_ref, e_ref, pbuf, sems):
    ii = pl.program_id(0)
    base = ii * RPB

    def issue(slot, rowbase):
        for mi in range(RPB):
            t = jnp.minimum(rowbase + mi, B * S - 1)
            pltpu.make_async_copy(p_any.at[ridx_s[t]],
                                  pbuf.at[slot * RPB + mi],
                                  sems.at[slot]).start()

    @pl.when(ii == 0)
    def _():
        for d in range(NSLOT - 1):
            issue(d, base + d * RPB)

    slot = jax.lax.rem(ii, NSLOT)

    @pl.when(ii < GK1 - (NSLOT - 1))
    def _():
        issue(jax.lax.rem(ii + NSLOT - 1, NSLOT), base + (NSLOT - 1) * RPB)

    for mi in range(RPB):
        t = base + mi
        e_ref[mi, 0] = ee_ref[et_s[t], 0] + ve_ref[vt_s[t], 0]

    sl = pl.ds(slot * RPB, RPB)
    pltpu.make_async_copy(pbuf.at[sl], pbuf.at[sl], sems.at[slot]).wait()
    r_ref[...] = pbuf[sl].astype(jnp.bfloat16)


def _run_k1(ridx, etf, vtf, P3, EE, VE):
    grid_spec = pltpu.PrefetchScalarGridSpec(
        num_scalar_prefetch=3,
        grid=(GK1,),
        in_specs=[
            pl.BlockSpec(memory_space=pl.ANY),
            pl.BlockSpec((NT + 1, 1, DM), lambda i, r, e, v: (0, 0, 0)),
            pl.BlockSpec((NV + 1, 1, DM), lambda i, r, e, v: (0, 0, 0)),
        ],
        out_specs=[
            pl.BlockSpec((RPB, 1, NV), lambda i, r, e, v: (i, 0, 0)),
            pl.BlockSpec((RPB, 1, DM), lambda i, r, e, v: (i, 0, 0)),
        ],
        scratch_shapes=[
            pltpu.VMEM((NSLOT * RPB, 1, NV), jnp.float32),
            pltpu.SemaphoreType.DMA((NSLOT,)),
        ],
    )
    return pl.pallas_call(
        _k1_body,
        grid_spec=grid_spec,
        out_shape=[
            jax.ShapeDtypeStruct((B * S, 1, NV), jnp.bfloat16),
            jax.ShapeDtypeStruct((B * S, 1, DM), jnp.float32),
        ],
        compiler_params=pltpu.CompilerParams(
            dimension_semantics=("arbitrary",),
            vmem_limit_bytes=56 * 1024 * 1024,
        ),
        name="k1_row_gather",
    )(ridx, etf, vtf, P3, EE, VE)


TI2 = 512
NBLK2 = S // TI2


def _k2_body(vert_ref, vmr_ref, r_ref, sim_ref, oh_ref):
    i = pl.program_id(1)

    @pl.when(i == 0)
    def _():
        idxr = jnp.maximum(vert_ref[0] - 1, 0)
        cio = jax.lax.broadcasted_iota(jnp.int32, (NV, S), 0)
        oh_ref[...] = jnp.where(cio == idxr, 1.0, 0.0).astype(jnp.bfloat16)

    simb = jnp.dot(r_ref[0], oh_ref[...], preferred_element_type=jnp.float32)
    colmask = jnp.where(vert_ref[0] != 0, 1.0, 0.0)
    sim_ref[0] = simb * (colmask * vmr_ref[0])


def _run_k2(vert3, vmcol, Rb):
    return pl.pallas_call(
        _k2_body,
        grid=(B, NBLK2),
        in_specs=[
            pl.BlockSpec((1, 1, S), lambda b, i: (b, 0, 0)),
            pl.BlockSpec((1, TI2, 1), lambda b, i: (b, i, 0)),
            pl.BlockSpec((1, TI2, NV), lambda b, i: (b, i, 0)),
        ],
        out_specs=pl.BlockSpec((1, TI2, S), lambda b, i: (b, i, 0)),
        out_shape=jax.ShapeDtypeStruct((B, S, S), jnp.float32),
        scratch_shapes=[pltpu.VMEM((NV, S), jnp.bfloat16)],
        compiler_params=pltpu.CompilerParams(
            dimension_semantics=("arbitrary", "arbitrary"),
            vmem_limit_bytes=56 * 1024 * 1024,
        ),
        name="k2_sim",
    )(vert3, vmcol, Rb)


def _k3_body(enc_full_ref, sim_ref, et_ref, t_ref, np_ref, npb_ref,
             wq_ref, wk_ref, wv_ref, wo_ref, bo_ref,
             g1_ref, b1l_ref, w1_ref, b1_ref, w2_ref, b2_ref,
             g2_ref, b2l_ref, ipv_ref, odd_ref,
             out_ref, ef_s, k_s, v_s):
    i = pl.program_id(1)

    @pl.when(i == 0)
    def _():
        tcol = t_ref[0]
        npcol = np_ref[0]
        arg = tcol * ipv_ref[...] + odd_ref[...]
        tem = _sin_poly(arg) * npcol
        ef = enc_full_ref[0] + tem
        ef_s[...] = ef
        eb = ef.astype(jnp.bfloat16)
        k_s[...] = jnp.dot(eb, wk_ref[...],
                           preferred_element_type=jnp.float32).astype(jnp.bfloat16)
        v_s[...] = jnp.dot(eb, wv_ref[...],
                           preferred_element_type=jnp.float32).astype(jnp.bfloat16)

    x = ef_s[pl.ds(i * TI3, TI3), :]
    q = jnp.dot(x.astype(jnp.bfloat16), wq_ref[...],
                preferred_element_type=jnp.float32)
    qb = (q * (1.0 / math.sqrt(DK))).astype(jnp.bfloat16)

    colg = jax.lax.broadcasted_iota(jnp.int32, (TI3, S), 1)
    rowg = jax.lax.broadcasted_iota(jnp.int32, (TI3, S), 0) + i * TI3
    mask = jnp.logical_or(et_ref[0] == 0, colg > rowg)
    simbias = jnp.where(mask, -1e9, sim_ref[0])

    outs = []
    for h in range(H):
        hs = slice(h * DK, (h + 1) * DK)
        sh = jax.lax.dot_general(
            qb[:, hs], k_s[:, hs],
            (((1,), (1,)), ((), ())),
            preferred_element_type=jnp.float32,
        ) + simbias
        m = jnp.max(sh, axis=-1, keepdims=True)
        p = jnp.exp(sh - m)
        l = jnp.sum(p, axis=-1, keepdims=True)
        oh = jnp.dot(p.astype(jnp.bfloat16), v_s[:, hs],
                     preferred_element_type=jnp.float32)
        outs.append(oh * (1.0 / l))
    o = jnp.concatenate(outs, axis=-1)
    o = jnp.dot(o.astype(jnp.bfloat16), wo_ref[...],
                preferred_element_type=jnp.float32) + bo_ref[...]

    npc = npb_ref[0]
    x1 = o + x
    mu = jnp.mean(x1, axis=-1, keepdims=True)
    d = x1 - mu
    var = jnp.mean(d * d, axis=-1, keepdims=True)
    y = d * jax.lax.rsqrt(var + 1e-5) * g1_ref[...] + b1l_ref[...]
    y = y * npc

    h1 = jnp.dot(y.astype(jnp.bfloat16), w1_ref[...],
                 preferred_element_type=jnp.float32) + b1_ref[...]
    h1 = jnp.maximum(h1, 0.0)
    f = jnp.dot(h1.astype(jnp.bfloat16), w2_ref[...],
                preferred_element_type=jnp.float32) + b2_ref[...]
    x2 = f + y
    mu2 = jnp.mean(x2, axis=-1, keepdims=True)
    d2 = x2 - mu2
    var2 = jnp.mean(d2 * d2, axis=-1, keepdims=True)
    z = d2 * jax.lax.rsqrt(var2 + 1e-5) * g2_ref[...] + b2l_ref[...]
    out_ref[0] = z * npc


def _run_k3(enc3, sim, et3, t3, np3, weights):
    (wq, wk, wv, wo, bo, g1, b1l, w1, b1, w2, b2, g2, b2l, ipv, odd) = weights
    full = lambda b, i: (b, 0, 0)
    blk = lambda b, i: (b, i, 0)
    zz = lambda b, i: (0, 0)
    return pl.pallas_call(
        _k3_body,
        grid=(B, NBLK3),
        in_specs=[
            pl.BlockSpec((1, S, DM), full),
            pl.BlockSpec((1, TI3, S), blk),
            pl.BlockSpec((1, 1, S), full),
            pl.BlockSpec((1, S, 1), full),
            pl.BlockSpec((1, S, 1), full),
            pl.BlockSpec((1, TI3, 1), blk),
            pl.BlockSpec((DM, DM), zz),
            pl.BlockSpec((DM, DM), zz),
            pl.BlockSpec((DM, DM), zz),
            pl.BlockSpec((DM, DM), zz),
            pl.BlockSpec((1, DM), zz),
            pl.BlockSpec((1, DM), zz),
            pl.BlockSpec((1, DM), zz),
            pl.BlockSpec((DM, DI), zz),
            pl.BlockSpec((1, DI), zz),
            pl.BlockSpec((DI, DM), zz),
            pl.BlockSpec((1, DM), zz),
            pl.BlockSpec((1, DM), zz),
            pl.BlockSpec((1, DM), zz),
            pl.BlockSpec((1, DM), zz),
            pl.BlockSpec((1, DM), zz),
        ],
        out_specs=pl.BlockSpec((1, TI3, DM), blk),
        out_shape=jax.ShapeDtypeStruct((B, S, DM), jnp.float32),
        scratch_shapes=[
            pltpu.VMEM((S, DM), jnp.float32),
            pltpu.VMEM((S, DM), jnp.bfloat16),
            pltpu.VMEM((S, DM), jnp.bfloat16),
        ],
        compiler_params=pltpu.CompilerParams(
            dimension_semantics=("parallel", "arbitrary"),
            vmem_limit_bytes=56 * 1024 * 1024,
        ),
        name="k3_encoder",
    )(enc3, sim, et3, t3, np3, np3,
      wq, wk, wv, wo, bo, g1, b1l, w1, b1, w2, b2, g2, b2l, ipv, odd)


def kernel(event_type, vertex, event_time, non_pad_mask, A, W, event_emb,
           vertex_emb, Wq, Wk, Wv, Wo, bo, ln1_g, ln1_b, w1, b1, w2, b2,
           ln2_g, ln2_b):
    et = event_type.astype(jnp.int32)
    vt = vertex.astype(jnp.int32)
    ridx = jnp.clip(vt - 1, 0).reshape(-1)
    etf = et.reshape(-1)
    vtf = vt.reshape(-1)
    A3 = A.reshape(NV, 1, NV)
    W3 = W.reshape(NV, 1, NV)
    EE = event_emb.reshape(NT + 1, 1, DM)
    VE = vertex_emb.reshape(NV + 1, 1, DM)

    P3 = _run_k0(A3, W3)
    R, enc0 = _run_k1(ridx, etf, vtf, P3, EE, VE)

    vert3 = vt.reshape(B, 1, S)
    vmcol = (vt != 0).astype(jnp.float32).reshape(B, S, 1)
    sim = _run_k2(vert3, vmcol, R.reshape(B, S, NV))

    iarr = jnp.arange(DM)
    pv = jnp.power(jnp.float32(10000.0),
                   2.0 * (iarr // 2).astype(jnp.float32) / DM)
    ipv = (1.0 / pv).reshape(1, DM)
    odd = jnp.where(iarr % 2 == 0, 0.0, _HALF_PI).astype(
        jnp.float32).reshape(1, DM)

    bf = jnp.bfloat16
    weights = (Wq.astype(bf), Wk.astype(bf), Wv.astype(bf), Wo.astype(bf),
               bo.reshape(1, DM), ln1_g.reshape(1, DM), ln1_b.reshape(1, DM),
               w1.astype(bf), b1.reshape(1, DI), w2.astype(bf),
               b2.reshape(1, DM), ln2_g.reshape(1, DM), ln2_b.reshape(1, DM),
               ipv, odd)

    et3 = et.reshape(B, 1, S)
    t3 = event_time.astype(jnp.float32).reshape(B, S, 1)
    np3 = (et != 0).astype(jnp.float32).reshape(B, S, 1)
    enc = _run_k3(enc0.reshape(B, S, DM), sim, et3, t3, np3, weights)

    return enc, sim.reshape(B, 1, S, S)
